# baseline (speedup 1.0000x reference)
import functools

import jax
import jax.numpy as jnp
from jax import lax
from jax.experimental import pallas as pl
from jax.experimental.pallas import tpu as pltpu
from jax.experimental.pallas import tpu_sc as plsc

_VOCAB = 1000000
_EMBED_DIM = 32
_BATCH = 16384

_NC = 2
_NS = 16
_NW = _NC * _NS
_COLS_PER_W = 244
_CCOLS = 6
_CHUNK = _CCOLS * 128
_NCHUNK = 42
_MAX_OFF = 999168
_TAIL_OFF = 999936
_LIST = _BATCH + 16
_CPACK = _BATCH + 32
_DUMP = _BATCH


def _stream_body(idx_hbm, table_hbm, out_hbm, idx_v, list_r, list_i,
                 cpack, chunk_a, chunk_b, val_v, rowidx_v,
                 sem_a, sem_b, sem_s):
    wid = lax.axis_index("s") * _NC + lax.axis_index("c")
    iota = lax.iota(jnp.int32, 16)
    cstart = wid * _COLS_PER_W + jnp.minimum(wid, 5)
    lo = cstart * 128
    hi = jnp.minimum((cstart + _COLS_PER_W + jnp.where(wid < 5, 1, 0)) * 128,
                     _VOCAB)

    pltpu.sync_copy(idx_hbm, idx_v)

    def scan_step(v, cnt):
        r = idx_v[pl.ds(v * 16, 16)]
        m = (r >= lo) & (r < hi)
        plsc.store_compressed(list_r.at[pl.ds(cnt, 16)], r, mask=m)
        plsc.store_compressed(list_i.at[pl.ds(cnt, 16)], v * 16 + iota,
                              mask=m)
        return cnt + jnp.max(plsc.all_reduce_population_count(m))

    cnt = lax.fori_loop(0, _BATCH // 16, scan_step, jnp.int32(0))
    list_r[pl.ds(cnt, 16)] = jnp.full((16,), -1, jnp.int32)
    list_i[pl.ds(cnt, 16)] = _DUMP + iota
    ntrip = (cnt + 15) // 16

    def chunk_off(c):
        return pl.multiple_of(
            jnp.minimum((cstart + c * _CCOLS) * 128, _MAX_OFF), 128)

    def fire_chunk(c, buf, sem):
        pltpu.async_copy(table_hbm.at[:, pl.ds(chunk_off(c), _CHUNK)],
                         buf, sem)

    def drain_chunk(sem):
        pltpu.make_async_copy(table_hbm.at[:, pl.ds(0, _CHUNK)],
                              chunk_a, sem).wait()

    def drain_scatter():
        pltpu.make_async_copy(out_hbm.at[pl.ds(0, 16)], val_v, sem_s).wait()

    def process(buf, clo, mlo, chi):
        def pair_step(w, ccnt):
            rv = list_r[pl.ds(w * 16, 16)]
            iv = list_i[pl.ds(w * 16, 16)]
            m = (rv >= mlo) & (rv < chi)
            plsc.store_compressed(cpack.at[pl.ds(ccnt, 16)],
                                  (iv << 10) | (rv - clo), mask=m)
            return ccnt + jnp.max(plsc.all_reduce_population_count(m))

        ccnt = lax.fori_loop(0, ntrip, pair_step, jnp.int32(0))
        cpack[pl.ds(ccnt, 16)] = ((_DUMP + iota) << 10)

        def wave_step(w, carry):
            pk = cpack[pl.ds(w * 16, 16)]
            rel = pk & 1023
            drain_scatter()
            rowidx_v[...] = pk >> 10
            for d in range(_EMBED_DIM):
                vals = plsc.load_gather(
                    buf, [jnp.full((16,), d, jnp.int32), rel])
                plsc.store_scatter(
                    val_v, [iota, jnp.full((16,), d, jnp.int32)], vals)
            pltpu.async_copy(val_v, out_hbm.at[rowidx_v], sem_s)
            return carry

        lax.fori_loop(0, (ccnt + 15) // 16, wave_step, jnp.int32(0))

    rowidx_v[...] = _DUMP + iota
    pltpu.async_copy(val_v, out_hbm.at[rowidx_v], sem_s)
    fire_chunk(0, chunk_a, sem_a)

    def pipe_step(j, carry):
        c0 = j * 2
        fire_chunk(c0 + 1, chunk_b, sem_b)
        drain_chunk(sem_a)
        clo0 = chunk_off(c0)
        process(chunk_a, clo0, jnp.maximum(clo0, lo), clo0 + _CHUNK)
        fire_chunk(c0 + 2, chunk_a, sem_a)
        drain_chunk(sem_b)
        clo1 = chunk_off(c0 + 1)
        process(chunk_b, clo1, jnp.maximum(clo1, lo), clo1 + _CHUNK)
        return carry

    lax.fori_loop(0, _NCHUNK // 2, pipe_step, jnp.int32(0))
    drain_chunk(sem_a)

    tail = pl.multiple_of(wid * 0 + _TAIL_OFF, 128)
    pltpu.sync_copy(table_hbm.at[:, pl.ds(tail, 128)],
                    chunk_a.at[:, pl.ds(0, 128)])
    process(chunk_a, tail, jnp.maximum(tail, lo), tail + 128)
    drain_scatter()


@jax.jit
def _stream_gather(idx, table_t):
    mesh = plsc.VectorSubcoreMesh(core_axis_name="c", subcore_axis_name="s")
    run = functools.partial(
        pl.kernel,
        out_type=jax.ShapeDtypeStruct((_BATCH + 16, 128), jnp.float32),
        mesh=mesh,
        scratch_types=[
            pltpu.VMEM((_BATCH,), jnp.int32),
            pltpu.VMEM((_LIST,), jnp.int32),
            pltpu.VMEM((_LIST,), jnp.int32),
            pltpu.VMEM((_CPACK,), jnp.int32),
            pltpu.VMEM((_EMBED_DIM, _CHUNK), jnp.float32),
            pltpu.VMEM((_EMBED_DIM, _CHUNK), jnp.float32),
            pltpu.VMEM((16, 128), jnp.float32),
            pltpu.VMEM((16,), jnp.int32),
            pltpu.SemaphoreType.DMA,
            pltpu.SemaphoreType.DMA,
            pltpu.SemaphoreType.DMA,
        ],
        compiler_params=pltpu.CompilerParams(
            needs_layout_passes=False, disable_bounds_checks=True),
    )(_stream_body)
    return run(idx, table_t)


def kernel(selector, kernel):
    idx = jnp.reshape(selector, (-1,)).astype(jnp.int32)
    table_t = jnp.transpose(kernel)
    out_p = _stream_gather(idx, table_t)
    return out_p[:_BATCH, :_EMBED_DIM]


# speedup vs baseline: 1.1372x; 1.1372x over previous
import functools

import jax
import jax.numpy as jnp
from jax import lax
from jax.experimental import pallas as pl
from jax.experimental.pallas import tpu as pltpu
from jax.experimental.pallas import tpu_sc as plsc

_VOCAB = 1000000
_EMBED_DIM = 32
_BATCH = 16384

_NC = 2
_NS = 16
_COLS_PER_W = 245
_CCOLS = 11
_CHUNK = _CCOLS * 128
_NCHUNK = 24
_MAX_OFF = 998528
_TAIL_OFF = 999936
_IDXBLK = 4096
_LIST = _BATCH + 64
_DUMP = _BATCH
_SENT = 32767


def _stream_body(idx_hbm, table_hbm, out_hbm, idxblk_v, lpack, cpack,
                 chunk_a, chunk_b, val_v, rowidx_v, sem_a, sem_b, sem_s):
    wid = lax.axis_index("s") * _NC + lax.axis_index("c")
    iota = lax.iota(jnp.int32, 16)
    cstart = wid * _COLS_PER_W
    lo = cstart * 128
    hi = jnp.minimum((cstart + _COLS_PER_W) * 128, _VOCAB)

    cnt_vec = jnp.int32(0)
    for blk in range(_BATCH // _IDXBLK):
        pltpu.sync_copy(idx_hbm.at[pl.ds(blk * _IDXBLK, _IDXBLK)], idxblk_v)

        def scan_step(v, cnt, blk=blk):
            for u in range(4):
                r = idxblk_v[pl.ds((v * 4 + u) * 16, 16)]
                m = (r >= lo) & (r < hi)
                pk = ((blk * _IDXBLK + (v * 4 + u) * 16 + iota) << 15) | (
                    r - lo)
                plsc.store_compressed(lpack.at[pl.ds(cnt, 16)], pk, mask=m)
                cnt = cnt + jnp.max(plsc.all_reduce_population_count(m))
            return cnt

        cnt_vec = lax.fori_loop(0, _IDXBLK // 64, scan_step, cnt_vec)
    cnt = cnt_vec
    for u in range(4):
        lpack[pl.ds(cnt + u * 16, 16)] = ((_DUMP + iota) << 15) | _SENT
    ntrip = (cnt + 63) // 64

    def chunk_off(c):
        return pl.multiple_of(
            jnp.minimum((cstart + c * _CCOLS) * 128, _MAX_OFF), 128)

    def fire_chunk(c, buf, sem):
        pltpu.async_copy(table_hbm.at[:, pl.ds(chunk_off(c), _CHUNK)],
                         buf, sem)

    def drain_chunk(sem):
        pltpu.make_async_copy(table_hbm.at[:, pl.ds(0, _CHUNK)],
                              chunk_a, sem).wait()

    def drain_scatter():
        pltpu.make_async_copy(out_hbm.at[pl.ds(0, 16)], val_v, sem_s).wait()

    def process(buf, clo, mlo, chi):
        rlo = mlo - lo
        rhi = chi - lo
        roff = clo - lo

        def pair_step(w, ccnt):
            for u in range(4):
                pk = lpack[pl.ds((w * 4 + u) * 16, 16)]
                rb = pk & _SENT
                m = (rb >= rlo) & (rb < rhi)
                plsc.store_compressed(cpack.at[pl.ds(ccnt, 16)], pk, mask=m)
                ccnt = ccnt + jnp.max(plsc.all_reduce_population_count(m))
            return ccnt

        ccnt = lax.fori_loop(0, ntrip, pair_step, jnp.int32(0))
        cpack[pl.ds(ccnt, 16)] = ((_DUMP + iota) << 15) | roff

        def wave_step(w, carry):
            pk = cpack[pl.ds(w * 16, 16)]
            rel = (pk & _SENT) - roff
            drain_scatter()
            rowidx_v[...] = pk >> 15
            for d in range(_EMBED_DIM):
                vals = plsc.load_gather(
                    buf, [jnp.full((16,), d, jnp.int32), rel])
                plsc.store_scatter(
                    val_v, [iota, jnp.full((16,), d, jnp.int32)], vals)
            pltpu.async_copy(val_v, out_hbm.at[rowidx_v], sem_s)
            return carry

        lax.fori_loop(0, (ccnt + 15) // 16, wave_step, jnp.int32(0))

    rowidx_v[...] = _DUMP + iota
    pltpu.async_copy(val_v, out_hbm.at[rowidx_v], sem_s)
    fire_chunk(0, chunk_a, sem_a)

    def pipe_step(j, carry):
        c0 = j * 2
        fire_chunk(c0 + 1, chunk_b, sem_b)
        drain_chunk(sem_a)
        clo0 = chunk_off(c0)
        process(chunk_a, clo0, jnp.maximum(clo0, lo), clo0 + _CHUNK)
        fire_chunk(c0 + 2, chunk_a, sem_a)
        drain_chunk(sem_b)
        clo1 = chunk_off(c0 + 1)
        process(chunk_b, clo1, jnp.maximum(clo1, lo), clo1 + _CHUNK)
        return carry

    lax.fori_loop(0, _NCHUNK // 2, pipe_step, jnp.int32(0))
    drain_chunk(sem_a)

    tail = pl.multiple_of(wid * 0 + _TAIL_OFF, 128)
    pltpu.sync_copy(table_hbm.at[:, pl.ds(tail, 128)],
                    chunk_a.at[:, pl.ds(0, 128)])
    process(chunk_a, tail, jnp.maximum(tail, lo), tail + 128)
    drain_scatter()


@jax.jit
def _stream_gather(idx, table_t):
    mesh = plsc.VectorSubcoreMesh(core_axis_name="c", subcore_axis_name="s")
    run = functools.partial(
        pl.kernel,
        out_type=jax.ShapeDtypeStruct((_BATCH + 16, 128), jnp.float32),
        mesh=mesh,
        scratch_types=[
            pltpu.VMEM((_IDXBLK,), jnp.int32),
            pltpu.VMEM((_LIST,), jnp.int32),
            pltpu.VMEM((_LIST,), jnp.int32),
            pltpu.VMEM((_EMBED_DIM, _CHUNK), jnp.float32),
            pltpu.VMEM((_EMBED_DIM, _CHUNK), jnp.float32),
            pltpu.VMEM((16, 128), jnp.float32),
            pltpu.VMEM((16,), jnp.int32),
            pltpu.SemaphoreType.DMA,
            pltpu.SemaphoreType.DMA,
            pltpu.SemaphoreType.DMA,
        ],
        compiler_params=pltpu.CompilerParams(
            needs_layout_passes=False, disable_bounds_checks=True),
    )(_stream_body)
    return run(idx, table_t)


def kernel(selector, kernel):
    idx = jnp.reshape(selector, (-1,)).astype(jnp.int32)
    table_t = jnp.transpose(kernel)
    out_p = _stream_gather(idx, table_t)
    return out_p[:_BATCH, :_EMBED_DIM]
